# Initial kernel scaffold; baseline (speedup 1.0000x reference)
import jax
import jax.numpy as jnp
from jax.experimental import pallas as pl


def kernel(x, edge_index, W1a, b1a, W2a, b2a, W1b, b1b, W2b, b2b):
    raise NotImplementedError("write your pallas kernel here")


# speedup vs baseline: 10.4220x; 10.4220x over previous
import functools

import jax
import jax.numpy as jnp
from jax import lax
from jax.experimental import pallas as pl
from jax.experimental.pallas import tpu as pltpu
from jax.experimental.pallas import tpu_sc as plsc

_NC = 2
_NS = 16
_CH = 128


def _sc_degree(np_rows, nch):
  cpw = nch // (_NC * _NS)
  rpt = np_rows // _NS
  mesh = plsc.VectorSubcoreMesh(core_axis_name="c", subcore_axis_name="s")

  def body(dst_hbm, ones_hbm, zeros_hbm, out_hbm, didx, ones_v, acc_sh):
    c = lax.axis_index("c")
    s = lax.axis_index("s")
    w = c * _NS + s
    pltpu.sync_copy(zeros_hbm.at[pl.ds(s * rpt, rpt)],
                    acc_sh.at[pl.ds(s * rpt, rpt)])
    pltpu.sync_copy(ones_hbm, ones_v)
    plsc.subcore_barrier()

    def step(i, carry):
      pltpu.sync_copy(dst_hbm.at[w * cpw + i], didx)
      pltpu.sync_copy(ones_v, acc_sh.at[didx], add=True)
      return carry

    lax.fori_loop(0, cpw, step, 0)
    plsc.subcore_barrier()
    pltpu.sync_copy(acc_sh.at[pl.ds(s * rpt, rpt)],
                    out_hbm.at[c, pl.ds(s * rpt, rpt)])

  return pl.kernel(
      body,
      out_type=jax.ShapeDtypeStruct((_NC, np_rows, 1), jnp.float32),
      mesh=mesh,
      scratch_types=[
          pltpu.VMEM((_CH,), jnp.int32),
          pltpu.VMEM((_CH, 1), jnp.float32),
          pltpu.VMEM_SHARED((np_rows, 1), jnp.float32),
      ],
  )


def _sc_scatter(np_rows, d, nch):
  cpw = nch // (_NC * _NS)
  rpt = np_rows // _NS
  mesh = plsc.VectorSubcoreMesh(core_axis_name="c", subcore_axis_name="s")

  def body(src_hbm, dst_hbm, table_hbm, zeros_hbm, out_hbm,
           sidx, didx, rows, acc_sh, gsem):
    c = lax.axis_index("c")
    s = lax.axis_index("s")
    w = c * _NS + s
    pltpu.sync_copy(zeros_hbm.at[pl.ds(s * rpt, rpt)],
                    acc_sh.at[pl.ds(s * rpt, rpt)])
    plsc.subcore_barrier()

    def step(i, carry):
      ch = w * cpw + i
      pltpu.sync_copy(src_hbm.at[ch], sidx)
      pltpu.sync_copy(dst_hbm.at[ch], didx)
      pltpu.async_copy(table_hbm.at[sidx], rows, gsem).wait()
      pltpu.sync_copy(rows, acc_sh.at[didx], add=True)
      return carry

    lax.fori_loop(0, cpw, step, 0)
    plsc.subcore_barrier()
    pltpu.sync_copy(acc_sh.at[pl.ds(s * rpt, rpt)],
                    out_hbm.at[c, pl.ds(s * rpt, rpt)])

  return pl.kernel(
      body,
      out_type=jax.ShapeDtypeStruct((_NC, np_rows, d), jnp.float32),
      mesh=mesh,
      scratch_types=[
          pltpu.VMEM((_CH,), jnp.int32),
          pltpu.VMEM((_CH,), jnp.int32),
          pltpu.VMEM((_CH, d), jnp.float32),
          pltpu.VMEM_SHARED((np_rows, d), jnp.float32),
          pltpu.SemaphoreType.DMA,
      ],
  )


def _tc_prep(np_rows, d, blk):
  def body(degp_ref, x_ref, xs_ref, dis_ref):
    deg = degp_ref[0] + degp_ref[1]
    dis = jnp.where(deg > 0.0, lax.rsqrt(jnp.maximum(deg, 1.0)), 0.0)
    xs_ref[...] = x_ref[...] * dis
    dis_ref[...] = dis

  return pl.pallas_call(
      body,
      grid=(np_rows // blk,),
      in_specs=[
          pl.BlockSpec((_NC, blk, 1), lambda i: (0, i, 0)),
          pl.BlockSpec((blk, d), lambda i: (i, 0)),
      ],
      out_specs=[
          pl.BlockSpec((blk, d), lambda i: (i, 0)),
          pl.BlockSpec((blk, 1), lambda i: (i, 0)),
      ],
      out_shape=[
          jax.ShapeDtypeStruct((np_rows, d), jnp.float32),
          jax.ShapeDtypeStruct((np_rows, 1), jnp.float32),
      ],
  )


def _tc_mid(np_rows, d, ncls, blk):
  def body(acc_ref, dis_ref, w1a_ref, b1a_ref, w1b_ref, b1b_ref,
           w2a_ref, w2b_ref, z_ref):
    dis = dis_ref[...]
    p = (acc_ref[0] + acc_ref[1]) * dis
    ha = jnp.maximum(
        jnp.dot(p, w1a_ref[...], preferred_element_type=jnp.float32)
        + b1a_ref[...], 0.0)
    hb = jnp.maximum(
        jnp.dot(p, w1b_ref[...], preferred_element_type=jnp.float32)
        + b1b_ref[...], 0.0)
    za = jnp.dot(ha, w2a_ref[...], preferred_element_type=jnp.float32)
    zb = jnp.dot(hb, w2b_ref[...], preferred_element_type=jnp.float32)
    z_ref[...] = jnp.concatenate([za, zb], axis=-1) * dis

  return pl.pallas_call(
      body,
      grid=(np_rows // blk,),
      in_specs=[
          pl.BlockSpec((_NC, blk, d), lambda i: (0, i, 0)),
          pl.BlockSpec((blk, 1), lambda i: (i, 0)),
          pl.BlockSpec((d, d), lambda i: (0, 0)),
          pl.BlockSpec((1, d), lambda i: (0, 0)),
          pl.BlockSpec((d, d), lambda i: (0, 0)),
          pl.BlockSpec((1, d), lambda i: (0, 0)),
          pl.BlockSpec((d, ncls), lambda i: (0, 0)),
          pl.BlockSpec((d, ncls), lambda i: (0, 0)),
      ],
      out_specs=pl.BlockSpec((blk, 2 * ncls), lambda i: (i, 0)),
      out_shape=jax.ShapeDtypeStruct((np_rows, 2 * ncls), jnp.float32),
  )


def _tc_final(np_rows, ncls, blk):
  def body(acc_ref, dis_ref, b2a_ref, b2b_ref, o1_ref, o2_ref):
    q = (acc_ref[0] + acc_ref[1]) * dis_ref[...]
    qa = q[:, :ncls] + b2a_ref[...]
    qb = q[:, ncls:] + b2b_ref[...]
    for qq, oref in ((qa, o1_ref), (qb, o2_ref)):
      m = jnp.max(qq, axis=-1, keepdims=True)
      lse = jnp.log(jnp.sum(jnp.exp(qq - m), axis=-1, keepdims=True))
      oref[...] = qq - m - lse

  return pl.pallas_call(
      body,
      grid=(np_rows // blk,),
      in_specs=[
          pl.BlockSpec((_NC, blk, 2 * ncls), lambda i: (0, i, 0)),
          pl.BlockSpec((blk, 1), lambda i: (i, 0)),
          pl.BlockSpec((1, ncls), lambda i: (0, 0)),
          pl.BlockSpec((1, ncls), lambda i: (0, 0)),
      ],
      out_specs=[
          pl.BlockSpec((blk, ncls), lambda i: (i, 0)),
          pl.BlockSpec((blk, ncls), lambda i: (i, 0)),
      ],
      out_shape=[
          jax.ShapeDtypeStruct((np_rows, ncls), jnp.float32),
          jax.ShapeDtypeStruct((np_rows, ncls), jnp.float32),
      ],
  )


@jax.jit
def kernel(x, edge_index, W1a, b1a, W2a, b2a, W1b, b1b, W2b, b2b):
  n, d = x.shape
  ncls = W2a.shape[1]
  e = edge_index.shape[1]

  np_rows = ((n + 1 + 127) // 128) * 128
  cpw = -(-e // (_CH * _NC * _NS))
  epad = cpw * _CH * _NC * _NS
  nch = epad // _CH

  src = jnp.concatenate(
      [edge_index[0], jnp.full((epad - e,), n, jnp.int32)]).reshape(nch, _CH)
  dst = jnp.concatenate(
      [edge_index[1], jnp.full((epad - e,), n, jnp.int32)]).reshape(nch, _CH)
  x_pad = jnp.pad(x, ((0, np_rows - n), (0, 0)))
  zeros_nd = jnp.zeros((np_rows, d), jnp.float32)
  zeros_n1 = jnp.zeros((np_rows, 1), jnp.float32)
  ones_ch = jnp.ones((_CH, 1), jnp.float32)

  blk = np_rows // 8
  scatter = _sc_scatter(np_rows, d, nch)

  deg_part = _sc_degree(np_rows, nch)(dst, ones_ch, zeros_n1)
  xs, dis = _tc_prep(np_rows, d, blk)(deg_part, x_pad)
  acc1 = scatter(src, dst, xs, zeros_nd)
  z = _tc_mid(np_rows, d, ncls, blk)(
      acc1, dis, W1a, b1a.reshape(1, d), W1b, b1b.reshape(1, d), W2a, W2b)
  acc2 = scatter(src, dst, z, zeros_nd)
  o1, o2 = _tc_final(np_rows, ncls, blk)(
      acc2, dis, b2a.reshape(1, ncls), b2b.reshape(1, ncls))
  return (o1[:n], o2[:n])


# speedup vs baseline: 13.1302x; 1.2598x over previous
import functools

import jax
import jax.numpy as jnp
from jax import lax
from jax.experimental import pallas as pl
from jax.experimental.pallas import tpu as pltpu
from jax.experimental.pallas import tpu_sc as plsc

_NC = 2
_NS = 16
_CH = 128


def _sc_degree(np_rows, nch):
  cpw = nch // (_NC * _NS)
  rpt = np_rows // _NS
  mesh = plsc.VectorSubcoreMesh(core_axis_name="c", subcore_axis_name="s")

  def body(dst_hbm, ones_hbm, zeros_hbm, out_hbm, didx, ones_v, acc_sh):
    c = lax.axis_index("c")
    s = lax.axis_index("s")
    w = c * _NS + s
    pltpu.sync_copy(zeros_hbm.at[pl.ds(s * rpt, rpt)],
                    acc_sh.at[pl.ds(s * rpt, rpt)])
    pltpu.sync_copy(ones_hbm, ones_v)
    plsc.subcore_barrier()

    def step(i, carry):
      pltpu.sync_copy(dst_hbm.at[w * cpw + i], didx)
      pltpu.sync_copy(ones_v, acc_sh.at[didx], add=True)
      return carry

    lax.fori_loop(0, cpw, step, 0)
    plsc.subcore_barrier()
    pltpu.sync_copy(acc_sh.at[pl.ds(s * rpt, rpt)],
                    out_hbm.at[c, pl.ds(s * rpt, rpt)])

  return pl.kernel(
      body,
      out_type=jax.ShapeDtypeStruct((_NC, np_rows, 1), jnp.float32),
      mesh=mesh,
      scratch_types=[
          pltpu.VMEM((_CH,), jnp.int32),
          pltpu.VMEM((_CH, 1), jnp.float32),
          pltpu.VMEM_SHARED((np_rows, 1), jnp.float32),
      ],
  )


def _sc_scatter(np_rows, d, nch):
  cpw = nch // (_NC * _NS)
  rpt = np_rows // _NS
  mesh = plsc.VectorSubcoreMesh(core_axis_name="c", subcore_axis_name="s")

  def body(src_hbm, dst_hbm, table_hbm, zeros_hbm, out_hbm,
           sidx0, didx0, rows0, sidx1, didx1, rows1, acc_sh, gsem0, gsem1):
    c = lax.axis_index("c")
    s = lax.axis_index("s")
    w = c * _NS + s
    base = w * cpw
    pltpu.sync_copy(zeros_hbm.at[pl.ds(s * rpt, rpt)],
                    acc_sh.at[pl.ds(s * rpt, rpt)])
    plsc.subcore_barrier()

    bufs = ((sidx0, didx0, rows0, gsem0), (sidx1, didx1, rows1, gsem1))

    pltpu.sync_copy(src_hbm.at[base], sidx0)
    pltpu.sync_copy(dst_hbm.at[base], didx0)
    g0 = pltpu.async_copy(table_hbm.at[sidx0], rows0, gsem0)

    def step(j, carry):
      for b in range(2):
        i = 2 * j + b
        sidx, didx, rows, gsem = bufs[b]
        nsidx, ndidx, nrows, ngsem = bufs[1 - b]
        nxt = i + 1

        @pl.when(nxt < cpw)
        def _():
          pltpu.sync_copy(src_hbm.at[base + nxt], nsidx)
          pltpu.sync_copy(dst_hbm.at[base + nxt], ndidx)
          pltpu.async_copy(table_hbm.at[nsidx], nrows, ngsem)

        @pl.when(i < cpw)
        def _():
          pltpu.make_async_copy(table_hbm.at[sidx], rows, gsem).wait()
          pltpu.sync_copy(rows, acc_sh.at[didx], add=True)
      return carry

    lax.fori_loop(0, (cpw + 1) // 2, step, 0)
    plsc.subcore_barrier()
    pltpu.sync_copy(acc_sh.at[pl.ds(s * rpt, rpt)],
                    out_hbm.at[c, pl.ds(s * rpt, rpt)])

  return pl.kernel(
      body,
      out_type=jax.ShapeDtypeStruct((_NC, np_rows, d), jnp.float32),
      mesh=mesh,
      scratch_types=[
          pltpu.VMEM((_CH,), jnp.int32),
          pltpu.VMEM((_CH,), jnp.int32),
          pltpu.VMEM((_CH, d), jnp.float32),
          pltpu.VMEM((_CH,), jnp.int32),
          pltpu.VMEM((_CH,), jnp.int32),
          pltpu.VMEM((_CH, d), jnp.float32),
          pltpu.VMEM_SHARED((np_rows, d), jnp.float32),
          pltpu.SemaphoreType.DMA,
          pltpu.SemaphoreType.DMA,
      ],
  )


def _tc_prep(np_rows, d, blk):
  def body(degp_ref, x_ref, xs_ref, dis_ref):
    deg = degp_ref[0] + degp_ref[1]
    dis = jnp.where(deg > 0.0, lax.rsqrt(jnp.maximum(deg, 1.0)), 0.0)
    xs_ref[...] = x_ref[...] * dis
    dis_ref[...] = dis

  return pl.pallas_call(
      body,
      grid=(np_rows // blk,),
      in_specs=[
          pl.BlockSpec((_NC, blk, 1), lambda i: (0, i, 0)),
          pl.BlockSpec((blk, d), lambda i: (i, 0)),
      ],
      out_specs=[
          pl.BlockSpec((blk, d), lambda i: (i, 0)),
          pl.BlockSpec((blk, 1), lambda i: (i, 0)),
      ],
      out_shape=[
          jax.ShapeDtypeStruct((np_rows, d), jnp.float32),
          jax.ShapeDtypeStruct((np_rows, 1), jnp.float32),
      ],
  )


def _tc_mid(np_rows, d, ncls, blk):
  def body(acc_ref, dis_ref, w1a_ref, b1a_ref, w1b_ref, b1b_ref,
           w2a_ref, w2b_ref, z_ref):
    dis = dis_ref[...]
    p = (acc_ref[0] + acc_ref[1]) * dis
    ha = jnp.maximum(
        jnp.dot(p, w1a_ref[...], preferred_element_type=jnp.float32)
        + b1a_ref[...], 0.0)
    hb = jnp.maximum(
        jnp.dot(p, w1b_ref[...], preferred_element_type=jnp.float32)
        + b1b_ref[...], 0.0)
    za = jnp.dot(ha, w2a_ref[...], preferred_element_type=jnp.float32)
    zb = jnp.dot(hb, w2b_ref[...], preferred_element_type=jnp.float32)
    z_ref[...] = jnp.concatenate([za, zb], axis=-1) * dis

  return pl.pallas_call(
      body,
      grid=(np_rows // blk,),
      in_specs=[
          pl.BlockSpec((_NC, blk, d), lambda i: (0, i, 0)),
          pl.BlockSpec((blk, 1), lambda i: (i, 0)),
          pl.BlockSpec((d, d), lambda i: (0, 0)),
          pl.BlockSpec((1, d), lambda i: (0, 0)),
          pl.BlockSpec((d, d), lambda i: (0, 0)),
          pl.BlockSpec((1, d), lambda i: (0, 0)),
          pl.BlockSpec((d, ncls), lambda i: (0, 0)),
          pl.BlockSpec((d, ncls), lambda i: (0, 0)),
      ],
      out_specs=pl.BlockSpec((blk, 2 * ncls), lambda i: (i, 0)),
      out_shape=jax.ShapeDtypeStruct((np_rows, 2 * ncls), jnp.float32),
  )


def _tc_final(np_rows, ncls, blk):
  def body(acc_ref, dis_ref, b2a_ref, b2b_ref, o1_ref, o2_ref):
    q = (acc_ref[0] + acc_ref[1]) * dis_ref[...]
    qa = q[:, :ncls] + b2a_ref[...]
    qb = q[:, ncls:] + b2b_ref[...]
    for qq, oref in ((qa, o1_ref), (qb, o2_ref)):
      m = jnp.max(qq, axis=-1, keepdims=True)
      lse = jnp.log(jnp.sum(jnp.exp(qq - m), axis=-1, keepdims=True))
      oref[...] = qq - m - lse

  return pl.pallas_call(
      body,
      grid=(np_rows // blk,),
      in_specs=[
          pl.BlockSpec((_NC, blk, 2 * ncls), lambda i: (0, i, 0)),
          pl.BlockSpec((blk, 1), lambda i: (i, 0)),
          pl.BlockSpec((1, ncls), lambda i: (0, 0)),
          pl.BlockSpec((1, ncls), lambda i: (0, 0)),
      ],
      out_specs=[
          pl.BlockSpec((blk, ncls), lambda i: (i, 0)),
          pl.BlockSpec((blk, ncls), lambda i: (i, 0)),
      ],
      out_shape=[
          jax.ShapeDtypeStruct((np_rows, ncls), jnp.float32),
          jax.ShapeDtypeStruct((np_rows, ncls), jnp.float32),
      ],
  )


@jax.jit
def kernel(x, edge_index, W1a, b1a, W2a, b2a, W1b, b1b, W2b, b2b):
  n, d = x.shape
  ncls = W2a.shape[1]
  e = edge_index.shape[1]

  np_rows = ((n + 1 + 127) // 128) * 128
  cpw = -(-e // (_CH * _NC * _NS))
  epad = cpw * _CH * _NC * _NS
  nch = epad // _CH

  src = jnp.concatenate(
      [edge_index[0], jnp.full((epad - e,), n, jnp.int32)]).reshape(nch, _CH)
  dst = jnp.concatenate(
      [edge_index[1], jnp.full((epad - e,), n, jnp.int32)]).reshape(nch, _CH)
  x_pad = jnp.pad(x, ((0, np_rows - n), (0, 0)))
  zeros_nd = jnp.zeros((np_rows, d), jnp.float32)
  zeros_n1 = jnp.zeros((np_rows, 1), jnp.float32)
  ones_ch = jnp.ones((_CH, 1), jnp.float32)

  blk = np_rows // 8
  scatter = _sc_scatter(np_rows, d, nch)

  deg_part = _sc_degree(np_rows, nch)(dst, ones_ch, zeros_n1)
  xs, dis = _tc_prep(np_rows, d, blk)(deg_part, x_pad)
  acc1 = scatter(src, dst, xs, zeros_nd)
  z = _tc_mid(np_rows, d, ncls, blk)(
      acc1, dis, W1a, b1a.reshape(1, d), W1b, b1b.reshape(1, d), W2a, W2b)
  acc2 = scatter(src, dst, z, zeros_nd)
  o1, o2 = _tc_final(np_rows, ncls, blk)(
      acc2, dis, b2a.reshape(1, ncls), b2b.reshape(1, ncls))
  return (o1[:n], o2[:n])
